# Initial kernel scaffold; baseline (speedup 1.0000x reference)
import jax
import jax.numpy as jnp
from jax.experimental import pallas as pl


def kernel(x, edge_attr, random_noise, edge_index, line_graph_edge_index, edge_index_batch, W_u, W_v, W_e, gcn_W, gcn_b, att1_W_rel, att1_b_rel, att1_W_root, a, lin_gout_W, lin_gout_b, a_bias, att2_W_rel, att2_b_rel, att2_W_root, c_W1g, c_b1g, c_W2g, c_b2g, c_W1n, c_b1n, c_W2n, c_b2n, W_out):
    raise NotImplementedError("write your pallas kernel here")


# speedup vs baseline: 3.4427x; 3.4427x over previous
import functools

import jax
import jax.numpy as jnp
from jax import lax
from jax.experimental import pallas as pl
from jax.experimental.pallas import tpu as pltpu
from jax.experimental.pallas import tpu_sc as plsc

F = 128
N_ = 10000
E_ = 160000
LGE_ = 320000
B_ = 64
NITER_ = 3
TAU_ = 0.5

NC, NS, L = 2, 16, 16
NW = NC * NS
SLAB = 10000
NSLAB = E_ // SLAB
GROWS = 64
PK = SLAB + GROWS
TPE = LGE_ // NW
AREA = 11072
CC = 64
FU = 16
MROW = 32
RB = 3200
NB = E_ // RB
NCH_E = E_ // 128
NCH_LG = LGE_ // 128

_mesh = plsc.VectorSubcoreMesh(
    core_axis_name="c", subcore_axis_name="s", num_cores=NC, num_subcores=NS)


def _wid():
    return lax.axis_index("c") * NS + lax.axis_index("s")


def _m8(v):
    return pl.multiple_of(v, 8)


def _zfill(ref, nvec, dtype=jnp.float32):
    z = jnp.zeros((16,), dtype)
    def body(i, _):
        ref[pl.ds(i * 16, 16)] = z
        return 0
    lax.fori_loop(0, nvec, body, 0)


def _pair_gather_body(eu, ev, src, dst, out, iu, iv, ru, rv, sem, sem2):
    w = _wid()
    nch = NCH_E // NW + jnp.where(w < (NCH_E % NW), 1, 0)

    def chunk(k, _):
        off = _m8((w + NW * k) * 128)
        pltpu.sync_copy(src.at[pl.ds(off, 128)], iu)
        pltpu.sync_copy(dst.at[pl.ds(off, 128)], iv)
        cu = pltpu.async_copy(eu.at[iu], ru, sem)
        cv = pltpu.async_copy(ev.at[iv], rv, sem2)
        cu.wait()
        cv.wait()

        def addrow(r, _):
            for j in range(8):
                ru[r, pl.ds(j * 16, 16)] = (
                    ru[r, pl.ds(j * 16, 16)] + rv[r, pl.ds(j * 16, 16)])
            return 0
        lax.fori_loop(0, 128, addrow, 0)
        pltpu.sync_copy(ru, out.at[pl.ds(off, 128)])
        return 0
    lax.fori_loop(0, nch, chunk, 0)


def _sc_pair_gather(eu, ev, src, dst):
    fn = pl.kernel(
        _pair_gather_body,
        out_type=jax.ShapeDtypeStruct((E_, F), jnp.float32),
        mesh=_mesh,
        scratch_types=[
            pltpu.VMEM((128,), jnp.int32),
            pltpu.VMEM((128,), jnp.int32),
            pltpu.VMEM((128, F), jnp.float32),
            pltpu.VMEM((128, F), jnp.float32),
            pltpu.SemaphoreType.DMA,
            pltpu.SemaphoreType.DMA,
        ],
    )
    return fn(eu, ev, src, dst)


def _scalar_conv_body(has_table, offs, *refs):
    if has_table:
        (table, ls, ld, out, acc, lsb, ldb, sbuf, valb, zb, sem) = refs
    else:
        (ld, out, acc, lsb, ldb, sbuf, valb, zb, sem) = refs
    cid = lax.axis_index("c")
    sid = lax.axis_index("s")
    stripe = E_ // NS
    _zfill(zb, 125)
    for q in range(5):
        pltpu.sync_copy(zb, acc.at[pl.ds(_m8(sid * stripe + q * 2000), 2000)])
    if not has_table:
        def of(i, _):
            valb[pl.ds(i * 16, 16)] = jnp.ones((16,), jnp.float32)
            return 0
        lax.fori_loop(0, 8, of, 0)
    plsc.subcore_barrier()

    half = NCH_LG // NC
    nch = half // NS + jnp.where(sid < (half % NS), 1, 0)

    def chunk(k, _):
        ci = cid * half + sid + NS * k
        off = _m8(ci * 128)
        pltpu.sync_copy(ld.at[pl.ds(off, 128)], ldb.at[0])
        if has_table:
            pltpu.sync_copy(ls.at[pl.ds(off, 128)], lsb)
            for j in range(8):
                sbuf[pl.ds(j * 16, 16)] = lsb[pl.ds(j * 16, 16)] + offs
            pltpu.async_copy(table.at[sbuf], valb, sem).wait()
        pltpu.sync_copy(valb, acc.at[ldb.at[0]], add=True)
        return 0
    lax.fori_loop(0, nch, chunk, 0)
    plsc.subcore_barrier()

    for q in range(5):
        o = _m8(sid * stripe + q * 2000)
        pltpu.sync_copy(acc.at[pl.ds(o, 2000)], zb)
        pltpu.sync_copy(zb, out.at[pl.ds(_m8(cid * E_ + sid * stripe + q * 2000), 2000)])


def _sc_scalar_conv(table_flat, ls, ld, offs):
    body = functools.partial(_scalar_conv_body, True, offs)
    fn = pl.kernel(
        body,
        out_type=jax.ShapeDtypeStruct((2 * E_,), jnp.float32),
        mesh=_mesh,
        scratch_types=[
            pltpu.VMEM_SHARED((E_,), jnp.float32),
            pltpu.VMEM((128,), jnp.int32),
            pltpu.VMEM((1, 128), jnp.int32),
            pltpu.VMEM((128,), jnp.int32),
            pltpu.VMEM((128,), jnp.float32),
            pltpu.VMEM((2000,), jnp.float32),
            pltpu.SemaphoreType.DMA,
        ],
    )
    out = fn(table_flat, ls, ld)
    return out[:E_], out[E_:]


def _sc_degree(ld):
    body = functools.partial(_scalar_conv_body, False, 0)
    fn = pl.kernel(
        body,
        out_type=jax.ShapeDtypeStruct((2 * E_,), jnp.float32),
        mesh=_mesh,
        scratch_types=[
            pltpu.VMEM_SHARED((E_,), jnp.float32),
            pltpu.VMEM((128,), jnp.int32),
            pltpu.VMEM((1, 128), jnp.int32),
            pltpu.VMEM((128,), jnp.int32),
            pltpu.VMEM((128,), jnp.float32),
            pltpu.VMEM((2000,), jnp.float32),
            pltpu.SemaphoreType.DMA,
        ],
    )
    out = fn(ld)
    return out[:E_], out[E_:]


_IOTA16 = lambda: lax.broadcasted_iota(jnp.int32, (16,), 0)


def _bucket_body(ls, ld, bpk, meta, lsb, ldb, ldb2, lsb2, flbuf, mrow, sm):
    w = _wid()
    base = _m8(w * TPE)

    def p1(k, _):
        pltpu.sync_copy(ld.at[pl.ds(_m8(base + k * 80), 80)], ldb)
        for j in range(5):
            lsb[pl.ds(j * 16, 16)] = lax.div(ldb[pl.ds(j * 16, 16)], SLAB)
        for j in range(5):
            sv = lsb[pl.ds(j * 16, 16)]
            for l in range(16):
                s = sv[l]
                sm[s] = sm[s] + 1
        return 0
    for s in range(NSLAB):
        sm[s] = 0
    lax.fori_loop(0, 125, p1, 0)

    run = w * AREA
    starts = jnp.zeros((16,), jnp.int32)
    lens = jnp.zeros((16,), jnp.int32)
    for s in range(NSLAB):
        plen = lax.div(sm[s] + (CC - 1), CC) * CC
        starts = jnp.where(_IOTA16() == s, run, starts)
        lens = jnp.where(_IOTA16() == s, plen, lens)
        sm[16 + s] = 0
        sm[32 + s] = run
        run = run + plen
    mrow[pl.ds(0, 16)] = starts
    mrow[pl.ds(16, 16)] = lens
    pltpu.sync_copy(mrow, meta.at[pl.ds(_m8(w * MROW), MROW)])

    def _flush16(s):
        fv = jnp.zeros((16,), jnp.int32)
        for j in range(16):
            fv = jnp.where(_IOTA16() == j, sm[64 + s * 32 + j], fv)
        flbuf[pl.ds(0, 16)] = fv
        pltpu.sync_copy(flbuf, bpk.at[pl.ds(_m8(sm[32 + s]), FU)])
        for j in range(15):
            sm[64 + s * 32 + j] = sm[64 + s * 32 + 16 + j]
        sm[32 + s] = sm[32 + s] + FU
        sm[16 + s] = sm[16 + s] - FU

    def p2(k, _):
        off = _m8(base + k * 16)
        pltpu.sync_copy(ls.at[pl.ds(off, 16)], lsb2)
        pltpu.sync_copy(ld.at[pl.ds(off, 16)], ldb2)
        vls = lsb2[pl.ds(0, 16)]
        vld = ldb2[pl.ds(0, 16)]
        svc = lax.div(vld, SLAB)
        lsb[pl.ds(0, 16)] = svc
        lsb[pl.ds(16, 16)] = vls * PK + (vld - svc * SLAB)
        sv = lsb[pl.ds(0, 16)]
        pkv = lsb[pl.ds(16, 16)]
        for l in range(16):
            s = sv[l]
            o = sm[16 + s]
            sm[64 + s * 32 + o] = pkv[l]
            sm[16 + s] = o + 1
        for s in range(NSLAB):
            @pl.when(sm[16 + s] >= FU)
            def _fl(s=s):
                _flush16(s)
        return 0
    lax.fori_loop(0, 625, p2, 0)

    for s in range(NSLAB):
        @pl.when(sm[16 + s] > 0)
        def _drain(s=s):
            o = sm[16 + s]
            for j in range(16):
                pad = (s * SLAB + j * 37) * PK + SLAB + j
                sm[64 + s * 32 + j] = jnp.where(
                    j >= o, pad, sm[64 + s * 32 + j])
            sm[16 + s] = FU
            _flush16(s)
        for _r in range(3):
            @pl.when(lax.rem(sm[32 + s], CC) != 0)
            def _pad(s=s):
                for j in range(16):
                    sm[64 + s * 32 + j] = (s * SLAB + j * 37) * PK + SLAB + j
                sm[16 + s] = FU
                _flush16(s)


def _sc_bucket(ls, ld):
    fn = pl.kernel(
        _bucket_body,
        out_type=(jax.ShapeDtypeStruct((NW * AREA,), jnp.int32),
                  jax.ShapeDtypeStruct((NW * MROW,), jnp.int32)),
        mesh=_mesh,
        scratch_types=[
            pltpu.VMEM((80,), jnp.int32),
            pltpu.VMEM((80,), jnp.int32),
            pltpu.VMEM((16,), jnp.int32),
            pltpu.VMEM((16,), jnp.int32),
            pltpu.VMEM((16,), jnp.int32),
            pltpu.VMEM((MROW,), jnp.int32),
            pltpu.SMEM((640,), jnp.int32),
        ],
    )
    return fn(ls, ld)


def _conv_body(xs, bpk, meta, out, acc, metab, pkb, sidx, didx, rows, zb,
               scm, sem_a, sem_b):
    cid = lax.axis_index("c")
    sid = lax.axis_index("s")

    def zf(r, _):
        for j in range(8):
            zb[r, pl.ds(j * 16, 16)] = jnp.zeros((16,), jnp.float32)
        return 0
    lax.fori_loop(0, 128, zf, 0)
    pltpu.sync_copy(meta.at[pl.ds(_m8((2 * sid) * MROW), MROW)],
                    metab.at[pl.ds(0, MROW)])
    pltpu.sync_copy(meta.at[pl.ds(_m8((2 * sid + 1) * MROW), MROW)],
                    metab.at[pl.ds(MROW, MROW)])
    for g in range(4):
        mv = metab[pl.ds(g * 16, 16)]
        for l in range(16):
            scm[g * 16 + l] = mv[l]

    def fire(k, start, par_sel):
        off = _m8(start + CC * k)

        def _issue(slot, sem):
            pltpu.sync_copy(bpk.at[pl.ds(off, CC)],
                            pkb.at[pl.ds(slot * CC, CC)])
            for j in range(4):
                v = pkb[pl.ds(slot * CC + j * 16, 16)]
                q = lax.div(v, PK)
                sidx[pl.ds(slot * CC + j * 16, 16)] = q
                didx[slot, pl.ds(j * 16, 16)] = v - q * PK
            pltpu.async_copy(xs.at[sidx.at[pl.ds(slot * CC, CC)]],
                             rows.at[pl.ds(slot * CC, CC)], sem)

        @pl.when(par_sel == 0)
        def _f0():
            _issue(0, sem_a)

        @pl.when(par_sel == 1)
        def _f1():
            _issue(1, sem_b)

    def slab_body(si, _):
        s = 2 * si + cid
        sbase = s * SLAB

        @pl.when(sid < 15)
        def _z0():
            for q in range(4):
                pltpu.sync_copy(zb, acc.at[pl.ds(_m8(sid * 632 + q * 128), 128)])
            pltpu.sync_copy(zb.at[pl.ds(0, 120)],
                            acc.at[pl.ds(_m8(sid * 632 + 512), 120)])

        @pl.when(sid == 15)
        def _z1():
            for q in range(4):
                pltpu.sync_copy(zb, acc.at[pl.ds(9480 + q * 128, 128)])
            pltpu.sync_copy(zb.at[pl.ds(0, 72)], acc.at[pl.ds(9992, 72)])
        plsc.subcore_barrier()

        for wl in range(2):
            start = scm[wl * MROW + s]
            nch = lax.div(scm[wl * MROW + 16 + s], CC)

            @pl.when(nch > 0)
            def _pro(start=start):
                fire(0, start, 0)

            def step(k, _, start=start, nch=nch):
                @pl.when(k + 1 < nch)
                def _nxt():
                    fire(k + 1, start, lax.rem(k + 1, 2))
                par = lax.rem(k, 2)

                @pl.when(par == 0)
                def _s0():
                    pltpu.make_async_copy(
                        xs.at[pl.ds(0, CC)], rows.at[pl.ds(0, CC)], sem_a).wait()
                    pltpu.sync_copy(rows.at[pl.ds(0, CC)], acc.at[didx.at[0]],
                                    add=True)

                @pl.when(par == 1)
                def _s1():
                    pltpu.make_async_copy(
                        xs.at[pl.ds(0, CC)], rows.at[pl.ds(CC, CC)], sem_b).wait()
                    pltpu.sync_copy(rows.at[pl.ds(CC, CC)], acc.at[didx.at[1]],
                                    add=True)
                return 0
            lax.fori_loop(0, nch, step, 0)
        plsc.subcore_barrier()

        @pl.when(sid < 10)
        def _wr():
            pltpu.sync_copy(acc.at[pl.ds(_m8(sid * 1000), 1000)],
                            out.at[pl.ds(_m8(sbase + sid * 1000), 1000)])
        plsc.subcore_barrier()
        return 0
    lax.fori_loop(0, NSLAB // NC, slab_body, 0)


def _sc_conv(xs, bpk, meta):
    fn = pl.kernel(
        _conv_body,
        out_type=jax.ShapeDtypeStruct((E_, F), jnp.float32),
        mesh=_mesh,
        scratch_types=[
            pltpu.VMEM_SHARED((SLAB + GROWS, F), jnp.float32),
            pltpu.VMEM((2 * MROW,), jnp.int32),
            pltpu.VMEM((2 * CC,), jnp.int32),
            pltpu.VMEM((2 * CC,), jnp.int32),
            pltpu.VMEM((2, CC), jnp.int32),
            pltpu.VMEM((2 * CC, F), jnp.float32),
            pltpu.VMEM((128, F), jnp.float32),
            pltpu.SMEM((64,), jnp.int32),
            pltpu.SemaphoreType.DMA,
            pltpu.SemaphoreType.DMA,
        ],
    )
    return fn(xs, bpk, meta)


def _dotf(a, b, hi=False):
    return jnp.dot(a, b, preferred_element_type=jnp.float32)


def _dot3(a, b):
    ah = a.astype(jnp.bfloat16).astype(jnp.float32)
    al = a - ah
    bh = b.astype(jnp.bfloat16).astype(jnp.float32)
    bl = b - bh

    def d(u, v):
        return jnp.dot(u, v, preferred_element_type=jnp.float32)
    return d(ah, bh) + d(al, bh) + d(ah, bl)


def _k_pre(x, wu, wv):
    def body(xb, wub, wvb, eo, vo):
        eo[...] = _dotf(xb[...], wub[...])
        vo[...] = _dotf(xb[...], wvb[...])
    return pl.pallas_call(
        body,
        grid=(N_ // RB,),
        in_specs=[pl.BlockSpec((RB, F), lambda i: (i, 0)),
                  pl.BlockSpec((F, F), lambda i: (0, 0)),
                  pl.BlockSpec((F, F), lambda i: (0, 0))],
        out_specs=[pl.BlockSpec((RB, F), lambda i: (i, 0)),
                   pl.BlockSpec((RB, F), lambda i: (i, 0))],
        out_shape=[jax.ShapeDtypeStruct((N_, F), jnp.float32),
                   jax.ShapeDtypeStruct((N_, F), jnp.float32)],
    )(x, wu, wv)


def _k_ea(pg, eattr, rn, we, w0, d0c, d1c):
    def body(pgb, eb, rnb, web, w0b, d0b, d1b, ea_o, nea_o, xse_o, xsn_o, dv_o):
        deg = 1.0 + d0b[0] + d1b[0]
        dinv = 1.0 / jnp.sqrt(deg)
        dv_o[0] = dinv
        ea = (pgb[...] + _dotf(eb[...], web[...])) * (1.0 / 3.0)
        ea_o[...] = ea
        r = rnb[...]
        nrm = jnp.maximum(jnp.sqrt(jnp.sum(r * r, axis=1, keepdims=True)), 1e-12)
        nea = ea + jnp.sign(ea) * (r / nrm) * 0.2
        nea_o[...] = nea
        xse_o[...] = _dotf(ea, w0b[...]) * dinv
        xsn_o[...] = _dotf(nea, w0b[...]) * dinv
    return pl.pallas_call(
        body,
        grid=(NB,),
        in_specs=[pl.BlockSpec((RB, F), lambda i: (i, 0)),
                  pl.BlockSpec((RB, 16), lambda i: (i, 0)),
                  pl.BlockSpec((RB, F), lambda i: (i, 0)),
                  pl.BlockSpec((16, F), lambda i: (0, 0)),
                  pl.BlockSpec((F, F), lambda i: (0, 0)),
                  pl.BlockSpec((1, RB, 1), lambda i: (i, 0, 0)),
                  pl.BlockSpec((1, RB, 1), lambda i: (i, 0, 0))],
        out_specs=[pl.BlockSpec((RB, F), lambda i: (i, 0)),
                   pl.BlockSpec((RB, F), lambda i: (i, 0)),
                   pl.BlockSpec((RB, F), lambda i: (i, 0)),
                   pl.BlockSpec((RB, F), lambda i: (i, 0)),
                   pl.BlockSpec((1, RB, 1), lambda i: (i, 0, 0))],
        out_shape=[jax.ShapeDtypeStruct((E_, F), jnp.float32),
                   jax.ShapeDtypeStruct((E_, F), jnp.float32),
                   jax.ShapeDtypeStruct((E_, F), jnp.float32),
                   jax.ShapeDtypeStruct((E_, F), jnp.float32),
                   jax.ShapeDtypeStruct((NB, RB, 1), jnp.float32)],
    )(pg, eattr, rn, we, w0, d0c, d1c)


def _k_iter(agg, xs, bbase, dinvc, bias, wnext, attv8, emit_xs):
    def body(ab, xb, bb, dvb, bib, wnb, avb, g_o, p_o, *rest):
        dinv = dvb[0]
        g = jax.nn.relu(bb[...] + (ab[...] + xb[...]) * dinv + bib[...])
        g_o[...] = g
        p_o[...] = lax.dot_general(
            avb[...], g, (((0,), (1,)), ((), ())),
            preferred_element_type=jnp.float32)
        if emit_xs:
            rest[0][...] = _dotf(g, wnb[...]) * dinv
    outs = [pl.BlockSpec((RB, F), lambda i: (i, 0)),
            pl.BlockSpec((8, RB), lambda i: (0, i))]
    oshapes = [jax.ShapeDtypeStruct((E_, F), jnp.float32),
               jax.ShapeDtypeStruct((8, E_), jnp.float32)]
    if emit_xs:
        outs.append(pl.BlockSpec((RB, F), lambda i: (i, 0)))
        oshapes.append(jax.ShapeDtypeStruct((E_, F), jnp.float32))
    return pl.pallas_call(
        functools.partial(body),
        grid=(NB,),
        in_specs=[pl.BlockSpec((RB, F), lambda i: (i, 0)),
                  pl.BlockSpec((RB, F), lambda i: (i, 0)),
                  pl.BlockSpec((RB, F), lambda i: (i, 0)),
                  pl.BlockSpec((1, RB, 1), lambda i: (i, 0, 0)),
                  pl.BlockSpec((1, F), lambda i: (0, 0)),
                  pl.BlockSpec((F, F), lambda i: (0, 0)),
                  pl.BlockSpec((F, 8), lambda i: (0, 0))],
        out_specs=outs,
        out_shape=oshapes,
    )(agg, xs, bbase, dinvc, bias, wnext, attv8)


def _k_segsoft(p0c, p1c, p2c, bscal, batchc):
    def body(p0b, p1b, p2b, bb, btb, m_o, d_o):
        i = pl.program_id(0)
        s = p0b[0] + p1b[0] + p2b[0] + bb[0, 0]
        maskT = btb[0] == lax.broadcasted_iota(jnp.int32, (RB, B_), 1)
        smT = jnp.where(maskT, s, -3e30)

        @pl.when(i == 0)
        def _init():
            m_o[...] = jnp.full((1, B_), -1e30, jnp.float32)
            d_o[...] = jnp.zeros((1, B_), jnp.float32)
        bmax = jnp.max(smT, axis=0, keepdims=True)
        m_old = m_o[...]
        m_new = jnp.maximum(m_old, bmax)
        d_o[...] = (d_o[...] * jnp.exp(m_old - m_new)
                    + jnp.sum(jnp.exp(smT - m_new), axis=0, keepdims=True))
        m_o[...] = m_new
    return pl.pallas_call(
        body,
        grid=(NB,),
        in_specs=[pl.BlockSpec((1, RB, 1), lambda i: (i, 0, 0)),
                  pl.BlockSpec((1, RB, 1), lambda i: (i, 0, 0)),
                  pl.BlockSpec((1, RB, 1), lambda i: (i, 0, 0)),
                  pl.BlockSpec((1, 1), lambda i: (0, 0)),
                  pl.BlockSpec((1, RB, 1), lambda i: (i, 0, 0))],
        out_specs=[pl.BlockSpec((1, B_), lambda i: (0, 0)),
                   pl.BlockSpec((1, B_), lambda i: (0, 0))],
        out_shape=[jax.ShapeDtypeStruct((1, B_), jnp.float32),
                   jax.ShapeDtypeStruct((1, B_), jnp.float32)],
    )(p0c, p1c, p2c, bscal, batchc)


def _k_gout(h, p0r, p1r, p2r, bscal, m, den, batchr):
    def body(hb, p0b, p1b, p2b, bb, mb, db, btb, go_o, bs_o):
        i = pl.program_id(0)
        s = p0b[0] + p1b[0] + p2b[0] + bb[0, 0]
        mask = (btb[0] == lax.broadcasted_iota(jnp.int32, (B_, RB), 0)
                ).astype(jnp.float32)
        msel = _dot3(mb[...], mask)
        dsel = _dot3(db[...], mask)
        sc = jnp.exp(s - msel) / (dsel + 1e-16)
        wmask = mask * sc

        @pl.when(i == 0)
        def _init():
            go_o[...] = jnp.zeros((B_, F), jnp.float32)
            bs_o[...] = jnp.zeros((B_, F), jnp.float32)
        hv = hb[...]
        go_o[...] = go_o[...] + _dot3(wmask, hv)
        bs_o[...] = bs_o[...] + _dot3(mask, hv)
    return pl.pallas_call(
        body,
        grid=(NB,),
        in_specs=[pl.BlockSpec((RB, F), lambda i: (i, 0)),
                  pl.BlockSpec((1, 1, RB), lambda i: (i, 0, 0)),
                  pl.BlockSpec((1, 1, RB), lambda i: (i, 0, 0)),
                  pl.BlockSpec((1, 1, RB), lambda i: (i, 0, 0)),
                  pl.BlockSpec((1, 1), lambda i: (0, 0)),
                  pl.BlockSpec((1, B_), lambda i: (0, 0)),
                  pl.BlockSpec((1, B_), lambda i: (0, 0)),
                  pl.BlockSpec((1, 1, RB), lambda i: (i, 0, 0))],
        out_specs=[pl.BlockSpec((B_, F), lambda i: (0, 0)),
                   pl.BlockSpec((B_, F), lambda i: (0, 0))],
        out_shape=[jax.ShapeDtypeStruct((B_, F), jnp.float32),
                   jax.ShapeDtypeStruct((B_, F), jnp.float32)],
    )(h, p0r, p1r, p2r, bscal, m, den, batchr)


def _k_scores(g0, g1, g2, b0, b1, b2, lw, lb, ac, abias):
    def body(g0b, g1b, g2b, b0b, b1b, b2b, lwb, lbb, acb, abb, sc_o, bo_o):
        eye = (lax.broadcasted_iota(jnp.int32, (B_, B_), 0)
               == lax.broadcasted_iota(jnp.int32, (B_, B_), 1)
               ).astype(jnp.float32)
        lts = []
        for gb, i in ((g0b, 0), (g1b, 1), (g2b, 2)):
            tg = jnp.tanh(_dotf(gb[...], lwb[...]) + lbb[...])
            lt = lax.dot_general(acb[...][:, i:i + 1], tg, (((0,), (1,)), ((), ())),
                                 preferred_element_type=jnp.float32)
            lts.append(lt + abb[...][:, i:i + 1])
        lg = jnp.concatenate(lts, axis=0)
        mx = jnp.max(lg, axis=0, keepdims=True)
        ex = jnp.exp(lg - mx)
        sm = ex / jnp.sum(ex, axis=0, keepdims=True)
        sc_o[...] = jnp.concatenate(
            [sm, jnp.zeros((5, B_), jnp.float32)], axis=0)
        bo = jnp.zeros((B_, F), jnp.float32)
        for bb, i in ((b0b, 0), (b1b, 1), (b2b, 2)):
            bo = bo + _dot3(eye * sm[i:i + 1, :], bb[...])
        bo_o[...] = bo
    full = lambda shp: pl.BlockSpec(shp, lambda: tuple(0 for _ in shp))
    return pl.pallas_call(
        body,
        in_specs=[full((B_, F))] * 6 + [full((F, F)), full((1, F)),
                                        full((F, NITER_)), full((1, NITER_))],
        out_specs=[full((8, B_)), full((B_, F))],
        out_shape=[jax.ShapeDtypeStruct((8, B_), jnp.float32),
                   jax.ShapeDtypeStruct((B_, F), jnp.float32)],
    )(g0, g1, g2, b0, b1, b2, lw, lb, ac, abias)


def _k_r2(p2r0, p2r1, p2r2, q2r0, q2r1, q2r2, scT, batchr):
    def body(a0, a1, a2, b0, b1, b2, sb, btb, o_o):
        mask = (btb[0] == lax.broadcasted_iota(jnp.int32, (B_, RB), 0)
                ).astype(jnp.float32)
        ses = [_dot3(sb[...][i:i + 1, :], mask) for i in range(3)]
        r2 = ses[0] * a0[0] + ses[1] * a1[0] + ses[2] * a2[0]
        q2 = ses[0] * b0[0] + ses[1] * b1[0] + ses[2] * b2[0]
        o_o[...] = jnp.concatenate(
            [r2, q2, jnp.zeros((6, RB), jnp.float32)], axis=0)
    return pl.pallas_call(
        body,
        grid=(NB,),
        in_specs=[pl.BlockSpec((1, 1, RB), lambda i: (i, 0, 0))] * 6
        + [pl.BlockSpec((8, B_), lambda i: (0, 0)),
           pl.BlockSpec((1, 1, RB), lambda i: (i, 0, 0))],
        out_specs=[pl.BlockSpec((8, RB), lambda i: (0, i))],
        out_shape=[jax.ShapeDtypeStruct((8, E_), jnp.float32)],
    )(p2r0, p2r1, p2r2, q2r0, q2r1, q2r2, scT, batchr)


def _k_final(h0, h1, h2, p0r, p1r, p2r, bscal, m, den, batchr, scT):
    def body(h0b, h1b, h2b, p0b, p1b, p2b, bb, mb, db, btb, scb,
             w0_o, w1_o, w2_o, bs_o):
        i = pl.program_id(0)
        s = p0b[0] + p1b[0] + p2b[0] + bb[0, 0]
        mask = (btb[0] == lax.broadcasted_iota(jnp.int32, (B_, RB), 0)
                ).astype(jnp.float32)
        msel = _dot3(mb[...], mask)
        dsel = _dot3(db[...], mask)
        sc = jnp.exp(s - msel) / (dsel + 1e-16)
        wmask = mask * sc

        @pl.when(i == 0)
        def _init():
            w0_o[...] = jnp.zeros((B_, F), jnp.float32)
            w1_o[...] = jnp.zeros((B_, F), jnp.float32)
            w2_o[...] = jnp.zeros((B_, F), jnp.float32)
            bs_o[...] = jnp.zeros((B_, F), jnp.float32)
        hv0, hv1, hv2 = h0b[...], h1b[...], h2b[...]
        w0_o[...] = w0_o[...] + _dot3(wmask, hv0)
        w1_o[...] = w1_o[...] + _dot3(wmask, hv1)
        w2_o[...] = w2_o[...] + _dot3(wmask, hv2)
        ses = [lax.dot_general(
            mask, scb[...][i2:i2 + 1, :], (((0,), (1,)), ((), ())),
            preferred_element_type=jnp.float32) for i2 in range(3)]
        orow = ses[0] * hv0 + ses[1] * hv1 + ses[2] * hv2
        orow = orow.astype(jnp.bfloat16).astype(jnp.float32)
        bs_o[...] = bs_o[...] + _dot3(mask, orow)
    return pl.pallas_call(
        body,
        grid=(NB,),
        in_specs=[pl.BlockSpec((RB, F), lambda i: (i, 0))] * 3
        + [pl.BlockSpec((1, 1, RB), lambda i: (i, 0, 0))] * 3
        + [pl.BlockSpec((1, 1), lambda i: (0, 0)),
           pl.BlockSpec((1, B_), lambda i: (0, 0)),
           pl.BlockSpec((1, B_), lambda i: (0, 0)),
           pl.BlockSpec((1, 1, RB), lambda i: (i, 0, 0)),
           pl.BlockSpec((8, B_), lambda i: (0, 0))],
        out_specs=[pl.BlockSpec((B_, F), lambda i: (0, 0))] * 4,
        out_shape=[jax.ShapeDtypeStruct((B_, F), jnp.float32)] * 4,
    )(h0, h1, h2, p0r, p1r, p2r, bscal, m, den, batchr, scT)


def _k_out(bo1, bo2, wa, wb, wc, wd, we_, wf, scT1, scT2, wout,
           w1g, b1g, w2g, b2g, w1n, b1n, w2n, b2n):
    def body(bo1b, bo2b, wab, wbb, wcb, wdb, web, wfb, s1b, s2b, wob,
             w1gb, b1gb, w2gb, b2gb, w1nb, b1nb, w2nb, b2nb, eg_o, el_o):
        eye = (lax.broadcasted_iota(jnp.int32, (B_, B_), 0)
               == lax.broadcasted_iota(jnp.int32, (B_, B_), 1)
               ).astype(jnp.float32)

        def comb(sb, ws):
            g = jnp.zeros((B_, F), jnp.float32)
            for i, w in enumerate(ws):
                g = g + _dotf(eye * sb[...][i:i + 1, :], w[...])
            return g
        g1 = comb(s1b, (wab, wbb, wcb))
        g2 = comb(s2b, (wdb, web, wfb))
        cat = jnp.concatenate([bo1b[...], bo2b[...]], axis=1)
        eg_o[...] = _dot3(cat, wob[...].astype(jnp.bfloat16).astype(jnp.float32))

        def elu(v):
            return jnp.where(v > 0, v, jnp.exp(jnp.minimum(v, 0.0)) - 1.0)
        h1 = _dotf(elu(_dotf(g1, w1gb[...]) + b1gb[...]), w2gb[...]) + b2gb[...]
        h2 = _dotf(elu(_dotf(g2, w1nb[...]) + b1nb[...]), w2nb[...]) + b2nb[...]
        h1 = h1 / jnp.maximum(
            jnp.sqrt(jnp.sum(h1 * h1, axis=1, keepdims=True)), 1e-8)
        h2 = h2 / jnp.maximum(
            jnp.sqrt(jnp.sum(h2 * h2, axis=1, keepdims=True)), 1e-8)
        sim = jnp.exp(lax.dot_general(h1, h2, (((1,), (1,)), ((), ())),
                                      preferred_element_type=jnp.float32) / TAU_)
        posr = jnp.sum(sim * eye, axis=0, keepdims=True)
        posc = jnp.sum(sim * eye, axis=1, keepdims=True)
        l1 = -jnp.log(posc / (jnp.sum(sim, axis=1, keepdims=True) + 1e-16))
        l2 = -jnp.log(posr / (jnp.sum(sim, axis=0, keepdims=True) + 1e-16))
        el_o[...] = (0.5 * (jnp.mean(l1) + jnp.mean(l2)))[None, None]
    full = lambda shp: pl.BlockSpec(shp, lambda: tuple(0 for _ in shp))
    return pl.pallas_call(
        body,
        in_specs=[full((B_, F))] * 8 + [full((8, B_))] * 2
        + [full((2 * F, F)), full((F, F)), full((1, F)), full((F, F)),
           full((1, F)), full((F, F)), full((1, F)), full((F, F)),
           full((1, F))],
        out_specs=[full((B_, F)), full((1, 1))],
        out_shape=[jax.ShapeDtypeStruct((B_, F), jnp.float32),
                   jax.ShapeDtypeStruct((1, 1), jnp.float32)],
    )(bo1, bo2, wa, wb, wc, wd, we_, wf, scT1, scT2, wout,
      w1g, b1g, w2g, b2g, w1n, b1n, w2n, b2n)


def _col(v):
    return v.reshape(NB, RB, 1)


def _row(v):
    return v.reshape(NB, 1, RB)


def kernel(x, edge_attr, random_noise, edge_index, line_graph_edge_index,
           edge_index_batch, W_u, W_v, W_e, gcn_W, gcn_b, att1_W_rel,
           att1_b_rel, att1_W_root, a, lin_gout_W, lin_gout_b, a_bias,
           att2_W_rel, att2_b_rel, att2_W_root, c_W1g, c_b1g, c_W2g, c_b2g,
           c_W1n, c_b1n, c_W2n, c_b2n, W_out):
    src = edge_index[0].astype(jnp.int32)
    dst = edge_index[1].astype(jnp.int32)
    ls = line_graph_edge_index[0].astype(jnp.int32)
    ld = line_graph_edge_index[1].astype(jnp.int32)
    batch = edge_index_batch.astype(jnp.int32)
    batchc, batchr = _col(batch), _row(batch)

    eu, ev = _k_pre(x, W_u, W_v)
    pg = _sc_pair_gather(eu, ev, src, dst)
    dg0, dg1 = _sc_degree(ld)
    bpk, meta = _sc_bucket(ls, ld)

    attv8 = jnp.concatenate(
        [att1_W_rel, att1_W_root, att2_W_rel, att2_W_root,
         jnp.zeros((F, 4), jnp.float32)], axis=1)
    b1s = att1_b_rel.reshape(1, 1)
    b2s = att2_b_rel.reshape(1, 1)

    ea, nea, xs0e, xs0n, dinvc = _k_ea(
        pg, edge_attr, random_noise, W_e, gcn_W[0], _col(dg0), _col(dg1))
    dinvc = dinvc.reshape(NB, RB, 1)

    def branch(bbase, xs0):
        xs_i = xs0
        hs, gouts, bsums, p2rs, q2rs = [], [], [], [], []
        for i in range(NITER_):
            agg = _sc_conv(xs_i, bpk, meta)
            wnext = gcn_W[(i + 1) % NITER_]
            outs = _k_iter(agg, xs_i, bbase, dinvc, gcn_b[i].reshape(1, F),
                           wnext, attv8, emit_xs=(i < NITER_ - 1))
            if i < NITER_ - 1:
                g, proj8, xs_i = outs
            else:
                g, proj8 = outs
            hs.append(g)
            pflat = proj8.reshape(-1)
            a1h0, a1h1 = _sc_scalar_conv(pflat, ls, ld, 0 * E_)
            q1 = proj8[1]
            m1, d1 = _k_segsoft(_col(a1h0), _col(a1h1), _col(q1), b1s, batchc)
            go, bs = _k_gout(g, _row(a1h0), _row(a1h1), _row(q1), b1s,
                             m1, d1, batchr)
            gouts.append(go)
            bsums.append(bs)
            p2rs.append(_row(proj8[2]))
            q2rs.append(_row(proj8[3]))
        scT, bsum_out = _k_scores(
            gouts[0], gouts[1], gouts[2], bsums[0], bsums[1], bsums[2],
            lin_gout_W, lin_gout_b.reshape(1, F), a.reshape(F, NITER_),
            a_bias.reshape(1, NITER_))
        r2q2 = _k_r2(p2rs[0], p2rs[1], p2rs[2], q2rs[0], q2rs[1], q2rs[2],
                     scT, batchr)[0]
        a2h0, a2h1 = _sc_scalar_conv(r2q2.reshape(-1), ls, ld, 0 * E_)
        qq2 = r2q2[1]
        m2, d2 = _k_segsoft(_col(a2h0), _col(a2h1), _col(qq2), b2s, batchc)
        w0, w1, w2, bsum_t = _k_final(hs[0], hs[1], hs[2], _row(a2h0),
                                      _row(a2h1), _row(qq2), b2s, m2, d2,
                                      batchr, scT)
        return bsum_t, (w0, w1, w2), scT

    bo1, ws1, scT1 = branch(ea, xs0e)
    bo2, ws2, scT2 = branch(nea, xs0n)
    e_g, e_loss = _k_out(bo1, bo2, ws1[0], ws1[1], ws1[2],
                         ws2[0], ws2[1], ws2[2], scT1, scT2, W_out,
                         c_W1g, c_b1g.reshape(1, F), c_W2g, c_b2g.reshape(1, F),
                         c_W1n, c_b1n.reshape(1, F), c_W2n, c_b2n.reshape(1, F))
    return (e_g, jnp.reshape(e_loss, ()))
